# Initial kernel scaffold; baseline (speedup 1.0000x reference)
import jax
import jax.numpy as jnp
from jax.experimental import pallas as pl


def kernel(z, pos, batch, emb, mlp_w1, mlp_b1, mlp_w2, mlp_b2, conv1_w, conv2_w, conv2_b, lin_w, lin_b, coord_w, coord_b, lin1_w, lin1_b, lin2_w, lin2_b):
    raise NotImplementedError("write your pallas kernel here")


# speedup vs baseline: 2.3351x; 2.3351x over previous
import functools

import jax
import jax.numpy as jnp
import numpy as np
from jax.experimental import pallas as pl
from jax.experimental.pallas import tpu as pltpu

_N_GAUSS = 50
_CUTOFF = 10.0
_MAXNB = 32
_BN = 128
_WIN = 512
_PL = 8
_LOG2 = 0.6931471805599453

_OFFS = np.linspace(0.0, _CUTOFF, _N_GAUSS).astype(np.float32)
_COEFF = float(-0.5 / (_OFFS[1] - _OFFS[0]) ** 2)


def _ssp(x):
    return jnp.log1p(jnp.exp(x)) - _LOG2


def _win_base(i, nwin):
    return jnp.clip((i * _BN) // _WIN - 1, 0, nwin - 2)


def _gather(idx, w0_ref, w1_ref, nwin):
    e = idx.shape[0]
    start = _win_base(pl.program_id(0), nwin) * _WIN
    rel = idx - start
    lanes = jax.lax.broadcasted_iota(jnp.int32, (e, _WIN), 1)
    oh0 = (rel == lanes).astype(jnp.float32)
    oh1 = ((rel - _WIN) == lanes).astype(jnp.float32)
    return (jnp.dot(oh0, w0_ref[...], preferred_element_type=jnp.float32, precision=jax.lax.Precision.HIGHEST)
            + jnp.dot(oh1, w1_ref[...], preferred_element_type=jnp.float32, precision=jax.lax.Precision.HIGHEST))


def _edge_geom(p_own, ps, offs):
    e = _BN * _MAXNB
    p_own_e = jnp.broadcast_to(p_own[:, None, :], (_BN, _MAXNB, _PL))
    p_own_e = p_own_e.reshape(e, _PL)
    rel = p_own_e - ps
    d2 = jnp.sum(rel * rel, axis=1, keepdims=True)
    ew = jnp.sqrt(d2 + 1e-12)
    d = ew - offs
    ea = jnp.exp(_COEFF * d * d)
    return rel, ew, ea


def _layer_a_kernel(hw0_ref, hw1_ref, pw0_ref, pw1_ref, h_own_ref, p_own_ref,
                    nb_ref, msk_ref, offs_ref, w1_ref, b1_ref, w2_ref, b2_ref,
                    c1_ref, c2_ref, c2b_ref, lw_ref, lb_ref, out_ref, *, nwin):
    e = _BN * _MAXNB
    idx = nb_ref[...]
    hs = _gather(idx, hw0_ref, hw1_ref, nwin)
    ps = _gather(idx, pw0_ref, pw1_ref, nwin)
    _, ew, ea = _edge_geom(p_own_ref[...], ps, offs_ref[0:1, :_N_GAUSS])
    f = _ssp(jnp.dot(ea, w1_ref[...], preferred_element_type=jnp.float32, precision=jax.lax.Precision.HIGHEST)
             + b1_ref[0:1, :])
    w = jnp.dot(f, w2_ref[...], preferred_element_type=jnp.float32, precision=jax.lax.Precision.HIGHEST) + b2_ref[0:1, :]
    env = 0.5 * (jnp.cos(ew * (np.pi / _CUTOFF)) + 1.0)
    w = w * env * msk_ref[...]
    xs = jnp.dot(hs, c1_ref[...], preferred_element_type=jnp.float32, precision=jax.lax.Precision.HIGHEST)
    agg = (xs * w).reshape(_BN, _MAXNB, 128).sum(axis=1)
    x = jnp.dot(agg, c2_ref[...], preferred_element_type=jnp.float32, precision=jax.lax.Precision.HIGHEST) + c2b_ref[0:1, :]
    x = _ssp(x)
    x = jnp.dot(x, lw_ref[...], preferred_element_type=jnp.float32, precision=jax.lax.Precision.HIGHEST) + lb_ref[0:1, :]
    out_ref[...] = h_own_ref[...] + x


def _layer_b_kernel(hw0_ref, hw1_ref, pw0_ref, pw1_ref, h_own_ref, p_own_ref,
                    nb_ref, msk_ref, offs_ref, cwd_ref, cws_ref, cwe_ref,
                    cb_ref, out_ref, *, nwin):
    e = _BN * _MAXNB
    idx = nb_ref[...]
    hs = _gather(idx, hw0_ref, hw1_ref, nwin)
    ps = _gather(idx, pw0_ref, pw1_ref, nwin)
    rel, _, ea = _edge_geom(p_own_ref[...], ps, offs_ref[0:1, :_N_GAUSS])
    h_own_e = jnp.broadcast_to(h_own_ref[...][:, None, :], (_BN, _MAXNB, 128))
    h_own_e = h_own_e.reshape(e, 128)
    w = (jnp.sum(h_own_e * cwd_ref[0:1, :], axis=1, keepdims=True)
         + jnp.sum(hs * cws_ref[0:1, :], axis=1, keepdims=True)
         + jnp.sum(ea * cwe_ref[0:1, :_N_GAUSS], axis=1, keepdims=True)
         + cb_ref[0:1, 0:1])
    msk = msk_ref[...]
    w = w * msk
    upd = (rel * w).reshape(_BN, _MAXNB, _PL).sum(axis=1)
    deg = msk.reshape(_BN, _MAXNB, 1).sum(axis=1)
    out_ref[...] = p_own_ref[...] + upd / (deg + 1.0)


def _readout_kernel(h_own_ref, bat_ref, l1w_ref, l1b_ref, l2w_ref, l2b_ref,
                    out_ref, *, ngraph):
    x = _ssp(jnp.dot(h_own_ref[...], l1w_ref[...],
                     preferred_element_type=jnp.float32, precision=jax.lax.Precision.HIGHEST) + l1b_ref[0:1, :])
    x = jnp.dot(x, l2w_ref[...], preferred_element_type=jnp.float32, precision=jax.lax.Precision.HIGHEST) + l2b_ref[0:1, 0:1]
    bat = bat_ref[...]
    gids = jax.lax.broadcasted_iota(jnp.int32, (_BN, ngraph), 1)
    onehot = (bat == gids).astype(jnp.float32)
    partial = jnp.sum(onehot * x, axis=0, keepdims=True)
    rows = jax.lax.broadcasted_iota(jnp.int32, (8, ngraph), 0)
    out_ref[...] = jnp.where(rows == 0, jnp.broadcast_to(partial, (8, ngraph)), 0.0)


def _full(arr):
    return pl.BlockSpec(arr.shape, lambda i: tuple(0 for _ in arr.shape))


def _rows(ncol):
    return pl.BlockSpec((_BN, ncol), lambda i: (i, 0))


def _wspec(ncol, nwin, off):
    return pl.BlockSpec((_WIN, ncol), lambda i: (_win_base(i, nwin) + off, 0))


def _pcall(kern, grid, in_specs, out_specs, out_shape):
    return pl.pallas_call(
        kern,
        grid=(grid,),
        in_specs=in_specs,
        out_specs=out_specs,
        out_shape=out_shape,
        compiler_params=pltpu.CompilerParams(
            dimension_semantics=("arbitrary",)),
    )


def kernel(z, pos, batch, emb, mlp_w1, mlp_b1, mlp_w2, mlp_b2, conv1_w,
           conv2_w, conv2_b, lin_w, lin_b, coord_w, coord_b, lin1_w, lin1_b,
           lin2_w, lin2_b):
    n = pos.shape[0]
    hidden = emb.shape[1]
    ni = mlp_w1.shape[0]
    ngraph = 512
    f32 = jnp.float32

    chunk = 2000
    nbs, vals = [], []
    for s in range(0, n, chunk):
        t = min(s + chunk, n)
        d2 = ((pos[s:t, None, :] - pos[None, :, :]) ** 2).sum(-1)
        d2 = jnp.where(batch[s:t, None] != batch[None, :], jnp.inf, d2)
        d2 = d2.at[jnp.arange(t - s), jnp.arange(s, t)].set(jnp.inf)
        negd, nbi = jax.lax.top_k(-d2, _MAXNB)
        nbs.append(nbi.astype(jnp.int32))
        vals.append((-negd) < _CUTOFF ** 2)
    nb = jnp.concatenate(nbs)
    valid = jnp.concatenate(vals)

    npad = max(2 * _WIN, ((n + _WIN - 1) // _WIN) * _WIN)
    pad = npad - n
    nwin = npad // _WIN
    h = emb[z].astype(f32)
    h = jnp.pad(h, ((0, pad), (0, 0)))
    p = jnp.pad(pos.astype(f32), ((0, pad), (0, _PL - 3)))
    nb_e = jnp.pad(nb, ((0, pad), (0, 0))).reshape(npad * _MAXNB, 1)
    msk_e = jnp.pad(valid.astype(f32), ((0, pad), (0, 0)))
    msk_e = msk_e.reshape(npad * _MAXNB, 1)
    bat = jnp.pad(batch.astype(jnp.int32), (0, pad),
                  constant_values=ngraph)[:, None]

    grid = npad // _BN
    espec = pl.BlockSpec((_BN * _MAXNB, 1), lambda i: (i, 0))

    def row1(x):
        return x.reshape(1, -1)

    hb = jax.ShapeDtypeStruct((npad, hidden), f32)
    pb = jax.ShapeDtypeStruct((npad, _PL), f32)
    offs2 = jnp.asarray(np.pad(_OFFS, (0, hidden - _N_GAUSS)).reshape(1, -1))

    def wins():
        return [_wspec(hidden, nwin, 0), _wspec(hidden, nwin, 1),
                _wspec(_PL, nwin, 0), _wspec(_PL, nwin, 1)]

    for i in range(ni):
        a_in = wins() + [_rows(hidden), _rows(_PL), espec, espec,
                         _full(offs2)]
        wargs = [mlp_w1[i], row1(mlp_b1[i]), mlp_w2[i], row1(mlp_b2[i]),
                 conv1_w[i], conv2_w[i], row1(conv2_b[i]), lin_w[i],
                 row1(lin_b[i])]
        a_in += [_full(w) for w in wargs]
        h = _pcall(functools.partial(_layer_a_kernel, nwin=nwin), grid,
                   a_in, _rows(hidden), hb)(
            h, h, p, p, h, p, nb_e, msk_e, offs2, *wargs)

        cwd = row1(coord_w[i][:hidden, 0])
        cws = row1(coord_w[i][hidden:2 * hidden, 0])
        cwe = row1(jnp.pad(coord_w[i][2 * hidden:, 0],
                           (0, hidden - _N_GAUSS)))
        cb = jnp.broadcast_to(coord_b[i].reshape(1, 1), (1, hidden))
        b_in = wins() + [_rows(hidden), _rows(_PL), espec, espec,
                         _full(offs2), _full(cwd), _full(cws), _full(cwe),
                         _full(cb)]
        p = _pcall(functools.partial(_layer_b_kernel, nwin=nwin), grid,
                   b_in, _rows(_PL), pb)(
            h, h, p, p, h, p, nb_e, msk_e, offs2, cwd, cws, cwe, cb)

    l2b2 = jnp.broadcast_to(lin2_b.reshape(1, 1), (1, hidden))
    l1b2 = row1(lin1_b)
    r_in = [_rows(hidden), pl.BlockSpec((_BN, 1), lambda i: (i, 0)),
            _full(lin1_w), _full(l1b2), _full(lin2_w), _full(l2b2)]
    partials = pl.pallas_call(
        functools.partial(_readout_kernel, ngraph=ngraph),
        grid=(grid,),
        in_specs=r_in,
        out_specs=pl.BlockSpec((8, ngraph), lambda i: (i, 0)),
        out_shape=jax.ShapeDtypeStruct((grid * 8, ngraph), f32),
        compiler_params=pltpu.CompilerParams(
            dimension_semantics=("arbitrary",)),
    )(h, bat, lin1_w, l1b2, lin2_w, l2b2)
    return partials.sum(axis=0)[:, None]
